# baseline (speedup 1.0000x reference)
import jax
import jax.numpy as jnp
from jax import lax
from jax.experimental import pallas as pl
from jax.experimental.pallas import tpu as pltpu

B, D, A, S = 16384, 128, 64, 64
BLK = 8192


def _tc_body(state_ref, wt_ref, bias_ref, wv_ref, bv_ref,
             mean_ref, value_ref, onehot_ref):
    x = state_ref[...]
    res = jnp.dot(x, wt_ref[...]) + bias_ref[...]
    mean_ref[...] = res[:, :A]
    value_ref[...] = lax.dot_general(
        wv_ref[...], x, (((1,), (1,)), ((), ()))) + bv_ref[...]
    logits = res[:, A:]
    idx = jnp.argmax(logits, axis=1)
    onehot_ref[...] = (
        lax.broadcasted_iota(jnp.int32, (BLK, S), 1) == idx[:, None]
    ).astype(jnp.float32)


@jax.jit
def kernel(state, W_skill, b_skill, W_action, b_action, W_value, b_value):
    wt = jnp.concatenate([W_action.T, W_skill.T], axis=1)
    bias = jnp.concatenate([b_action, b_skill])[None, :]

    grid = (B // BLK,)
    mean, value, one_hot = pl.pallas_call(
        _tc_body,
        grid=grid,
        in_specs=[
            pl.BlockSpec((BLK, D), lambda i: (i, 0)),
            pl.BlockSpec((D, 128), lambda i: (0, 0)),
            pl.BlockSpec((1, 128), lambda i: (0, 0)),
            pl.BlockSpec((1, D), lambda i: (0, 0)),
            pl.BlockSpec((1, 1), lambda i: (0, 0)),
        ],
        out_specs=[
            pl.BlockSpec((BLK, A), lambda i: (i, 0)),
            pl.BlockSpec((1, BLK), lambda i: (0, i)),
            pl.BlockSpec((BLK, S), lambda i: (i, 0)),
        ],
        out_shape=[
            jax.ShapeDtypeStruct((B, A), jnp.float32),
            jax.ShapeDtypeStruct((1, B), jnp.float32),
            jax.ShapeDtypeStruct((B, S), jnp.float32),
        ],
        compiler_params=pltpu.CompilerParams(
            dimension_semantics=("parallel",),
        ),
    )(state, wt, bias, W_value, b_value[None, :])
    std = jnp.zeros((B, A), jnp.float32)
    return (mean, std, value[0], one_hot)
